# baseline (speedup 1.0000x reference)
import jax
import jax.numpy as jnp
from jax.experimental import pallas as pl
from jax.experimental.pallas import tpu as pltpu

_BLOCK_ROWS = 1024


def _avg_kernel(x1_ref, x2_ref, o_ref):
    o_ref[...] = (x1_ref[...] + x2_ref[...]) * 0.5


def kernel(x_1, x_2):
    rows, cols = x_1.shape
    grid = (rows // _BLOCK_ROWS,)
    spec = pl.BlockSpec((_BLOCK_ROWS, cols), lambda i: (i, 0))
    return pl.pallas_call(
        _avg_kernel,
        grid=grid,
        in_specs=[spec, spec],
        out_specs=spec,
        out_shape=jax.ShapeDtypeStruct((rows, cols), x_1.dtype),
        compiler_params=pltpu.CompilerParams(
            dimension_semantics=("arbitrary",),
        ),
    )(x_1, x_2)


# speedup vs baseline: 1.0215x; 1.0215x over previous
import jax
import jax.numpy as jnp
from jax.experimental import pallas as pl
from jax.experimental.pallas import tpu as pltpu

_BLOCK_ROWS = 2048


def _avg_kernel(x1_ref, x2_ref, o_ref):
    o_ref[...] = (x1_ref[...] + x2_ref[...]) * 0.5


def kernel(x_1, x_2):
    rows, cols = x_1.shape
    grid = (rows // _BLOCK_ROWS,)
    spec = pl.BlockSpec((_BLOCK_ROWS, cols), lambda i: (i, 0))
    return pl.pallas_call(
        _avg_kernel,
        grid=grid,
        in_specs=[spec, spec],
        out_specs=spec,
        out_shape=jax.ShapeDtypeStruct((rows, cols), x_1.dtype),
        compiler_params=pltpu.CompilerParams(
            dimension_semantics=("arbitrary",),
        ),
    )(x_1, x_2)
